# baseline (speedup 1.0000x reference)
import functools

import jax
import jax.numpy as jnp
from jax import lax
from jax.experimental import pallas as pl
from jax.experimental.pallas import tpu as pltpu
from jax.experimental.pallas import tpu_sc as plsc

_DIM = 256
_NE = 8192
_NTOK = 8192
_BLK = 512


_W = 1024
_RT = 64
_LAN = 128


def _argmin_body(zb_ref, cb_ref, idx_ref, e2_ref):
    @pl.when(pl.program_id(0) == 0)
    def _():
        cb = cb_ref[...]
        e2_ref[...] = jnp.sum(cb * cb, axis=1).reshape(1, _NE)

    zraw = zb_ref[...].reshape(_DIM, _BLK)
    zb = zraw.T
    z2 = jnp.sum(zb * zb, axis=1, keepdims=True)
    zb2 = zb + zb
    nR = _BLK // _RT
    nW = _NE // _W
    nC = _W // _LAN
    z2r = [z2[r * _RT:(r + 1) * _RT, :] for r in range(nR)]
    bv = [None] * nR
    bi = [None] * nR
    for w in range(nW):
        cbw = cb_ref[pl.ds(w * _W, _W), :]
        pw = lax.dot_general(zb2, cbw, (((1,), (1,)), ((), ())),
                             preferred_element_type=jnp.float32)
        e2w = e2_ref[:, pl.ds(w * _W, _W)]
        for r in range(nR):
            for c in range(nC):
                p = pw[r * _RT:(r + 1) * _RT, c * _LAN:(c + 1) * _LAN]
                e2c = e2w[:, c * _LAN:(c + 1) * _LAN]
                d = (z2r[r] + e2c) - p
                vid = w * nC + c
                if vid == 0:
                    bv[r] = d
                    bi[r] = jnp.zeros(d.shape, jnp.int32)
                else:
                    lt = d < bv[r]
                    bv[r] = jnp.where(lt, d, bv[r])
                    bi[r] = jnp.where(lt, jnp.int32(vid), bi[r])
    lane = lax.broadcasted_iota(jnp.int32, (_RT, _LAN), 1)
    parts = []
    for r in range(nR):
        m = jnp.min(bv[r], axis=1, keepdims=True)
        g = bi[r] * _LAN + lane
        cand = jnp.where(bv[r] == m, g, jnp.int32(_NE))
        parts.append(jnp.min(cand, axis=1))
    idx_ref[...] = jnp.concatenate(parts)


def _build_argmin(interpret: bool = False):
    return pl.pallas_call(
        _argmin_body,
        grid=(_NTOK // _BLK,),
        in_specs=[
            pl.BlockSpec((1, _DIM, 16, 32), lambda i: (i // 2, 0, i % 2, 0)),
            pl.BlockSpec((_NE, _DIM), lambda i: (0, 0)),
        ],
        out_specs=pl.BlockSpec((_BLK,), lambda i: (i,)),
        out_shape=jax.ShapeDtypeStruct((_NTOK,), jnp.int32),
        scratch_shapes=[pltpu.VMEM((1, _NE), jnp.float32)],
        compiler_params=pltpu.CompilerParams(
            dimension_semantics=("arbitrary",)),
        interpret=interpret,
    )


_tc_argmin = _build_argmin()

_NC = 2
_NS = 16
_NW = _NC * _NS
_BPW = _NTOK // _NW
_CH = 128

@functools.cache
def _build_sc_gather():
    mesh = plsc.VectorSubcoreMesh(core_axis_name="c", subcore_axis_name="s",
                                  num_cores=_NC, num_subcores=_NS)

    @functools.partial(
        pl.kernel,
        out_type=jax.ShapeDtypeStruct((_NTOK, _DIM), jnp.float32),
        mesh=mesh,
        scratch_types=[
            pltpu.VMEM((_CH,), jnp.int32),
            pltpu.VMEM((_CH,), jnp.int32),
            pltpu.VMEM((_CH, _DIM), jnp.float32),
            pltpu.VMEM((_CH, _DIM), jnp.float32),
            pltpu.SemaphoreType.DMA,
        ],
    )
    def _sc_gather(cb_hbm, idx_hbm, out_hbm, idx_a, idx_b, rows_a, rows_b, sem):
        wid = lax.axis_index("s") * _NC + lax.axis_index("c")
        base = wid * _BPW
        pltpu.sync_copy(idx_hbm.at[pl.ds(base, _CH)], idx_a)
        pltpu.sync_copy(idx_hbm.at[pl.ds(base + _CH, _CH)], idx_b)
        cp0 = pltpu.async_copy(cb_hbm.at[idx_a], rows_a, sem)
        cp1 = pltpu.async_copy(cb_hbm.at[idx_b], rows_b, sem)
        cp0.wait()
        cp1.wait()
        pltpu.sync_copy(rows_a, out_hbm.at[pl.ds(base, _CH)])
        pltpu.sync_copy(rows_b, out_hbm.at[pl.ds(base + _CH, _CH)])

    return _sc_gather


def kernel(z, codebook):
    idx = _tc_argmin(z, codebook)
    z_q = _build_sc_gather()(codebook, idx)
    return z_q.reshape(z.shape[0], z.shape[2], z.shape[3], _DIM), idx


# speedup vs baseline: 1.3117x; 1.3117x over previous
import functools

import jax
import jax.numpy as jnp
from jax import lax
from jax.experimental import pallas as pl
from jax.experimental.pallas import tpu as pltpu
from jax.experimental.pallas import tpu_sc as plsc

_DIM = 256
_NE = 8192
_NTOK = 8192
_BLK = 512


_W = 2048
_RT = 64
_LAN = 128


def _argmin_body(zb_ref, cb_ref, idx_ref, e2_ref):
    @pl.when(pl.program_id(0) == 0)
    def _():
        cb = cb_ref[...]
        e2_ref[...] = jnp.sum(cb * cb, axis=1).reshape(1, _NE)

    zb = zb_ref[...]
    z2 = jnp.sum(zb * zb, axis=1, keepdims=True)
    zb2 = zb + zb
    nR = _BLK // _RT
    nW = _NE // _W
    nC = _W // _LAN
    z2r = [z2[r * _RT:(r + 1) * _RT, :] for r in range(nR)]
    bv = [None] * nR
    bi = [None] * nR
    for w in range(nW):
        cbw = cb_ref[pl.ds(w * _W, _W), :]
        pw = lax.dot_general(zb2, cbw, (((1,), (1,)), ((), ())),
                             preferred_element_type=jnp.float32)
        e2w = e2_ref[:, pl.ds(w * _W, _W)]
        for r in range(nR):
            for c in range(nC):
                p = pw[r * _RT:(r + 1) * _RT, c * _LAN:(c + 1) * _LAN]
                e2c = e2w[:, c * _LAN:(c + 1) * _LAN]
                d = (z2r[r] + e2c) - p
                vid = w * nC + c
                if vid == 0:
                    bv[r] = d
                    bi[r] = jnp.zeros(d.shape, jnp.int32)
                else:
                    lt = d < bv[r]
                    bv[r] = jnp.where(lt, d, bv[r])
                    bi[r] = jnp.where(lt, jnp.int32(vid), bi[r])
    lane = lax.broadcasted_iota(jnp.int32, (_RT, _LAN), 1)
    parts = []
    for r in range(nR):
        m = jnp.min(bv[r], axis=1, keepdims=True)
        g = bi[r] * _LAN + lane
        cand = jnp.where(bv[r] == m, g, jnp.int32(_NE))
        parts.append(jnp.min(cand, axis=1))
    idx_ref[...] = jnp.concatenate(parts)


def _build_argmin(interpret: bool = False):
    return pl.pallas_call(
        _argmin_body,
        grid=(_NTOK // _BLK,),
        in_specs=[
            pl.BlockSpec((_BLK, _DIM), lambda i: (i, 0)),
            pl.BlockSpec((_NE, _DIM), lambda i: (0, 0)),
        ],
        out_specs=pl.BlockSpec((_BLK,), lambda i: (i,)),
        out_shape=jax.ShapeDtypeStruct((_NTOK,), jnp.int32),
        scratch_shapes=[pltpu.VMEM((1, _NE), jnp.float32)],
        compiler_params=pltpu.CompilerParams(
            dimension_semantics=("arbitrary",)),
        interpret=interpret,
    )


_tc_argmin = _build_argmin()

_NC = 2
_NS = 16
_NW = _NC * _NS
_BPW = _NTOK // _NW
_CH = 128

@functools.cache
def _build_sc_gather():
    mesh = plsc.VectorSubcoreMesh(core_axis_name="c", subcore_axis_name="s",
                                  num_cores=_NC, num_subcores=_NS)

    @functools.partial(
        pl.kernel,
        out_type=jax.ShapeDtypeStruct((_NTOK, _DIM), jnp.float32),
        mesh=mesh,
        scratch_types=[
            pltpu.VMEM((_CH,), jnp.int32),
            pltpu.VMEM((_CH,), jnp.int32),
            pltpu.VMEM((_CH, _DIM), jnp.float32),
            pltpu.VMEM((_CH, _DIM), jnp.float32),
            pltpu.SemaphoreType.DMA,
        ],
    )
    def _sc_gather(cb_hbm, idx_hbm, out_hbm, idx_a, idx_b, rows_a, rows_b, sem):
        wid = lax.axis_index("s") * _NC + lax.axis_index("c")
        base = wid * _BPW
        pltpu.sync_copy(idx_hbm.at[pl.ds(base, _CH)], idx_a)
        pltpu.sync_copy(idx_hbm.at[pl.ds(base + _CH, _CH)], idx_b)
        cp0 = pltpu.async_copy(cb_hbm.at[idx_a], rows_a, sem)
        cp1 = pltpu.async_copy(cb_hbm.at[idx_b], rows_b, sem)
        cp0.wait()
        cp1.wait()
        pltpu.sync_copy(rows_a, out_hbm.at[pl.ds(base, _CH)])
        pltpu.sync_copy(rows_b, out_hbm.at[pl.ds(base + _CH, _CH)])

    return _sc_gather


def kernel(z, codebook):
    z_perm = jnp.transpose(z, (0, 2, 3, 1))
    z_flat = z_perm.reshape(-1, _DIM)
    idx = _tc_argmin(z_flat, codebook)
    z_q = _build_sc_gather()(codebook, idx)
    return z_q.reshape(z_perm.shape), idx


# speedup vs baseline: 1.3135x; 1.0014x over previous
import functools

import jax
import jax.numpy as jnp
from jax import lax
from jax.experimental import pallas as pl
from jax.experimental.pallas import tpu as pltpu
from jax.experimental.pallas import tpu_sc as plsc

_DIM = 256
_NE = 8192
_NTOK = 8192
_BLK = 512


_W = 512
_RT = 64
_LAN = 128


def _argmin_body(zb_ref, cb_ref, idx_ref, e2_ref):
    @pl.when(pl.program_id(0) == 0)
    def _():
        cb = cb_ref[...]
        e2_ref[...] = jnp.sum(cb * cb, axis=1).reshape(1, _NE)

    zb = zb_ref[...]
    z2 = jnp.sum(zb * zb, axis=1, keepdims=True)
    zb2 = zb + zb
    nR = _BLK // _RT
    nW = _NE // _W
    nC = _W // _LAN
    z2r = [z2[r * _RT:(r + 1) * _RT, :] for r in range(nR)]
    bv = [None] * nR
    bi = [None] * nR
    for w in range(nW):
        cbw = cb_ref[pl.ds(w * _W, _W), :]
        pw = lax.dot_general(zb2, cbw, (((1,), (1,)), ((), ())),
                             preferred_element_type=jnp.float32)
        e2w = e2_ref[:, pl.ds(w * _W, _W)]
        for r in range(nR):
            for c in range(nC):
                p = pw[r * _RT:(r + 1) * _RT, c * _LAN:(c + 1) * _LAN]
                e2c = e2w[:, c * _LAN:(c + 1) * _LAN]
                d = (z2r[r] + e2c) - p
                vid = w * nC + c
                if vid == 0:
                    bv[r] = d
                    bi[r] = jnp.zeros(d.shape, jnp.int32)
                else:
                    lt = d < bv[r]
                    bv[r] = jnp.where(lt, d, bv[r])
                    bi[r] = jnp.where(lt, jnp.int32(vid), bi[r])
    lane = lax.broadcasted_iota(jnp.int32, (_RT, _LAN), 1)
    parts = []
    for r in range(nR):
        m = jnp.min(bv[r], axis=1, keepdims=True)
        g = bi[r] * _LAN + lane
        cand = jnp.where(bv[r] == m, g, jnp.int32(_NE))
        parts.append(jnp.min(cand, axis=1))
    idx_ref[...] = jnp.concatenate(parts)


def _build_argmin(interpret: bool = False):
    return pl.pallas_call(
        _argmin_body,
        grid=(_NTOK // _BLK,),
        in_specs=[
            pl.BlockSpec((_BLK, _DIM), lambda i: (i, 0)),
            pl.BlockSpec((_NE, _DIM), lambda i: (0, 0)),
        ],
        out_specs=pl.BlockSpec((_BLK,), lambda i: (i,)),
        out_shape=jax.ShapeDtypeStruct((_NTOK,), jnp.int32),
        scratch_shapes=[pltpu.VMEM((1, _NE), jnp.float32)],
        compiler_params=pltpu.CompilerParams(
            dimension_semantics=("arbitrary",)),
        interpret=interpret,
    )


_tc_argmin = _build_argmin()

_NC = 2
_NS = 16
_NW = _NC * _NS
_BPW = _NTOK // _NW
_CH = 128

@functools.cache
def _build_sc_gather():
    mesh = plsc.VectorSubcoreMesh(core_axis_name="c", subcore_axis_name="s",
                                  num_cores=_NC, num_subcores=_NS)

    @functools.partial(
        pl.kernel,
        out_type=jax.ShapeDtypeStruct((_NTOK, _DIM), jnp.float32),
        mesh=mesh,
        scratch_types=[
            pltpu.VMEM((_CH,), jnp.int32),
            pltpu.VMEM((_CH,), jnp.int32),
            pltpu.VMEM((_CH, _DIM), jnp.float32),
            pltpu.VMEM((_CH, _DIM), jnp.float32),
            pltpu.SemaphoreType.DMA,
        ],
    )
    def _sc_gather(cb_hbm, idx_hbm, out_hbm, idx_a, idx_b, rows_a, rows_b, sem):
        wid = lax.axis_index("s") * _NC + lax.axis_index("c")
        base = wid * _BPW
        pltpu.sync_copy(idx_hbm.at[pl.ds(base, _CH)], idx_a)
        pltpu.sync_copy(idx_hbm.at[pl.ds(base + _CH, _CH)], idx_b)
        cp0 = pltpu.async_copy(cb_hbm.at[idx_a], rows_a, sem)
        cp1 = pltpu.async_copy(cb_hbm.at[idx_b], rows_b, sem)
        cp0.wait()
        cp1.wait()
        pltpu.sync_copy(rows_a, out_hbm.at[pl.ds(base, _CH)])
        pltpu.sync_copy(rows_b, out_hbm.at[pl.ds(base + _CH, _CH)])

    return _sc_gather


def kernel(z, codebook):
    z_perm = jnp.transpose(z, (0, 2, 3, 1))
    z_flat = z_perm.reshape(-1, _DIM)
    idx = _tc_argmin(z_flat, codebook)
    z_q = _build_sc_gather()(codebook, idx)
    return z_q.reshape(z_perm.shape), idx
